# baseline (speedup 1.0000x reference)
import functools

import jax
import jax.numpy as jnp
from jax import lax
from jax.experimental import pallas as pl
from jax.experimental.pallas import tpu as pltpu, tpu_sc as plsc

D = 64
NC = 2
NS = 16
NW = NC * NS

ROWS = 4096 * 200
ROWS_PER_W = ROWS // NW
CHUNK = 256
NCHUNK = ROWS_PER_W // CHUNK
L = 16
VPR = D // L


def _sc_body(x_hbm, perm_hbm, signs_hbm, out_hbm, perm_v, signs_v, in_v, out_v):
    wid = lax.axis_index("s") * NC + lax.axis_index("c")
    pltpu.sync_copy(perm_hbm, perm_v)
    pltpu.sync_copy(signs_hbm, signs_v)
    pvecs = [perm_v[pl.ds(L * j, L)] for j in range(VPR)]
    svecs = [signs_v[pl.ds(L * j, L)] for j in range(VPR)]
    base = wid * ROWS_PER_W * D

    def chunk_body(i, _):
        off = base + i * CHUNK * D
        pltpu.sync_copy(x_hbm.at[pl.ds(off, CHUNK * D)], in_v)

        def row_body(r, _):
            rb = r * D
            rbv = jnp.full((L,), rb, jnp.int32)
            for j in range(VPR):
                v = plsc.load_gather(in_v, [pvecs[j] + rbv])
                out_v[pl.ds(rb + L * j, L)] = v * svecs[j]
            return 0

        lax.fori_loop(0, CHUNK, row_body, 0, unroll=4)
        pltpu.sync_copy(out_v, out_hbm.at[pl.ds(off, CHUNK * D)])
        return 0

    lax.fori_loop(0, NCHUNK, chunk_body, 0)


@jax.jit
def kernel(x, perm, signs):
    mesh = plsc.VectorSubcoreMesh(
        core_axis_name="c", subcore_axis_name="s", num_cores=NC, num_subcores=NS
    )
    run = pl.kernel(
        _sc_body,
        out_type=jax.ShapeDtypeStruct((ROWS * D,), jnp.float32),
        mesh=mesh,
        scratch_types=[
            pltpu.VMEM((D,), jnp.int32),
            pltpu.VMEM((D,), jnp.float32),
            pltpu.VMEM((CHUNK * D,), jnp.float32),
            pltpu.VMEM((CHUNK * D,), jnp.float32),
        ],
        compiler_params=pltpu.CompilerParams(needs_layout_passes=False),
    )
    out_flat = run(x.reshape(-1), perm, signs)
    return out_flat.reshape(x.shape)


# speedup vs baseline: 1.5335x; 1.5335x over previous
import jax
import jax.numpy as jnp
from jax import lax
from jax.experimental import pallas as pl
from jax.experimental.pallas import tpu as pltpu, tpu_sc as plsc

D = 64
NC = 2
NS = 16
NW = NC * NS

ROWS = 4096 * 200
ROWS_PER_W = ROWS // NW
CHUNK = 400
NCHUNK = ROWS_PER_W // CHUNK
NBUF = 2
L = 16
VPR = D // L


def _sc_body(x_hbm, perm_hbm, signs_hbm, out_hbm,
             perm_v, signs_v, in_v0, in_v1, out_v0, out_v1,
             in_sem0, in_sem1, out_sem0, out_sem1):
    in_bufs = [in_v0, in_v1]
    out_bufs = [out_v0, out_v1]
    in_sems = [in_sem0, in_sem1]
    out_sems = [out_sem0, out_sem1]
    wid = lax.axis_index("s") * NC + lax.axis_index("c")
    pltpu.sync_copy(perm_hbm, perm_v)
    pltpu.sync_copy(signs_hbm, signs_v)
    pvecs = [perm_v[pl.ds(L * j, L)] for j in range(VPR)]
    svecs = [signs_v[pl.ds(L * j, L)] for j in range(VPR)]
    step64 = jnp.full((L,), D, jnp.int32)
    base = wid * ROWS_PER_W * D

    def in_slice(i):
        return x_hbm.at[pl.ds(base + i * CHUNK * D, CHUNK * D)]

    def out_slice(i):
        return out_hbm.at[pl.ds(base + i * CHUNK * D, CHUNK * D)]

    for b in range(NBUF):
        pltpu.async_copy(in_slice(b), in_bufs[b], in_sems[b])

    def compute(b):
        def row_body(r, idx):
            rb = r * D
            nidx = []
            for j in range(VPR):
                v = plsc.load_gather(in_bufs[b], [idx[j]])
                out_bufs[b][pl.ds(rb + L * j, L)] = v * svecs[j]
                nidx.append(idx[j] + step64)
            return tuple(nidx)

        plsc.parallel_loop(0, CHUNK, 1, unroll=8, carry=tuple(pvecs))(row_body)

    def chunk_pair(i2, _):
        for b in range(NBUF):
            i = i2 * NBUF + b
            pltpu.make_async_copy(in_slice(i), in_bufs[b], in_sems[b]).wait()
            @pl.when(i2 > 0)
            def _():
                pltpu.make_async_copy(
                    out_bufs[b], out_slice(i - NBUF), out_sems[b]).wait()
            compute(b)
            pltpu.async_copy(out_bufs[b], out_slice(i), out_sems[b])
            @pl.when(i + NBUF < NCHUNK)
            def _():
                pltpu.async_copy(in_slice(i + NBUF), in_bufs[b], in_sems[b])
        return 0

    lax.fori_loop(0, NCHUNK // NBUF, chunk_pair, 0)
    for b in range(NBUF):
        pltpu.make_async_copy(
            out_bufs[b], out_slice(NCHUNK - NBUF + b), out_sems[b]).wait()


@jax.jit
def kernel(x, perm, signs):
    mesh = plsc.VectorSubcoreMesh(
        core_axis_name="c", subcore_axis_name="s", num_cores=NC, num_subcores=NS
    )
    run = pl.kernel(
        _sc_body,
        out_type=jax.ShapeDtypeStruct((ROWS * D,), jnp.float32),
        mesh=mesh,
        scratch_types=[
            pltpu.VMEM((D,), jnp.int32),
            pltpu.VMEM((D,), jnp.float32),
            pltpu.VMEM((CHUNK * D,), jnp.float32),
            pltpu.VMEM((CHUNK * D,), jnp.float32),
            pltpu.VMEM((CHUNK * D,), jnp.float32),
            pltpu.VMEM((CHUNK * D,), jnp.float32),
            pltpu.SemaphoreType.DMA,
            pltpu.SemaphoreType.DMA,
            pltpu.SemaphoreType.DMA,
            pltpu.SemaphoreType.DMA,
        ],
        compiler_params=pltpu.CompilerParams(needs_layout_passes=False),
    )
    out_flat = run(x.reshape(-1), perm, signs)
    return out_flat.reshape(x.shape)


# speedup vs baseline: 1.9527x; 1.2734x over previous
import jax
import jax.numpy as jnp
from jax import lax
from jax.experimental import pallas as pl
from jax.experimental.pallas import tpu as pltpu, tpu_sc as plsc

B = 4096
T = 200
D = 64
NC = 2
NS = 16
NW = NC * NS

B_PER_W = B // NW
NBUF = 2
L = 16
VPR = D // L


def _sc_body(x_hbm, perm_hbm, signs_hbm, out_hbm,
             perm_v, signs_v, in_v0, in_v1, out_v0, out_v1,
             in_sem0, in_sem1, out_sem0, out_sem1):
    in_bufs = [in_v0, in_v1]
    out_bufs = [out_v0, out_v1]
    in_sems = [in_sem0, in_sem1]
    out_sems = [out_sem0, out_sem1]
    wid = lax.axis_index("s") * NC + lax.axis_index("c")
    pltpu.sync_copy(perm_hbm, perm_v)
    pltpu.sync_copy(signs_hbm, signs_v)
    pvecs = [perm_v[pl.ds(L * j, L)] for j in range(VPR)]
    svecs = [signs_v[pl.ds(L * j, L)] for j in range(VPR)]
    one = jnp.full((L,), 1, jnp.int32)
    base = wid * B_PER_W

    for b in range(NBUF):
        pltpu.async_copy(x_hbm.at[base + b], in_bufs[b], in_sems[b])

    def compute(b):
        def row_body(t, rowvec):
            for j in range(VPR):
                v = plsc.load_gather(in_bufs[b], [rowvec, pvecs[j]])
                out_bufs[b][t, pl.ds(L * j, L)] = v * svecs[j]
            return rowvec + one

        plsc.parallel_loop(0, T, 1, unroll=8,
                           carry=jnp.full((L,), 0, jnp.int32))(row_body)

    def chunk_pair(i2, _):
        for b in range(NBUF):
            i = base + i2 * NBUF + b
            pltpu.make_async_copy(x_hbm.at[i], in_bufs[b], in_sems[b]).wait()
            @pl.when(i2 > 0)
            def _():
                pltpu.make_async_copy(
                    out_bufs[b], out_hbm.at[i - NBUF], out_sems[b]).wait()
            compute(b)
            pltpu.async_copy(out_bufs[b], out_hbm.at[i], out_sems[b])
            @pl.when(i2 * NBUF + b + NBUF < B_PER_W)
            def _():
                pltpu.async_copy(x_hbm.at[i + NBUF], in_bufs[b], in_sems[b])
        return 0

    lax.fori_loop(0, B_PER_W // NBUF, chunk_pair, 0)
    for b in range(NBUF):
        pltpu.make_async_copy(
            out_bufs[b], out_hbm.at[base + B_PER_W - NBUF + b],
            out_sems[b]).wait()


@jax.jit
def kernel(x, perm, signs):
    mesh = plsc.VectorSubcoreMesh(
        core_axis_name="c", subcore_axis_name="s", num_cores=NC, num_subcores=NS
    )
    run = pl.kernel(
        _sc_body,
        out_type=jax.ShapeDtypeStruct((B, T, D), jnp.float32),
        mesh=mesh,
        scratch_types=[
            pltpu.VMEM((D,), jnp.int32),
            pltpu.VMEM((D,), jnp.float32),
            pltpu.VMEM((T, D), jnp.float32),
            pltpu.VMEM((T, D), jnp.float32),
            pltpu.VMEM((T, D), jnp.float32),
            pltpu.VMEM((T, D), jnp.float32),
            pltpu.SemaphoreType.DMA,
            pltpu.SemaphoreType.DMA,
            pltpu.SemaphoreType.DMA,
            pltpu.SemaphoreType.DMA,
        ],
        compiler_params=pltpu.CompilerParams(
            needs_layout_passes=False,
            use_tc_tiling_on_sc=True,
        ),
    )
    return run(x, perm, signs)


# speedup vs baseline: 8.5730x; 4.3904x over previous
import jax
import jax.numpy as jnp
from jax import lax
from jax.experimental import pallas as pl
from jax.experimental.pallas import tpu as pltpu, tpu_sc as plsc

B = 4096
T = 200
D = 64
NC = 2
NS = 16
NW = NC * NS

CB = 128
NUNITS = T * (B // CB)
U_PER_W = NUNITS // NW
NBUF = 2
L = 16


def _sc_body(x_hbm, perm_hbm, signs_hbm, out_hbm,
             perm_v, signs_v, in_v0, in_v1, out_v0, out_v1,
             in_sem0, in_sem1, out_sem0, out_sem1):
    in_bufs = [in_v0, in_v1]
    out_bufs = [out_v0, out_v1]
    in_sems = [in_sem0, in_sem1]
    out_sems = [out_sem0, out_sem1]
    wid = lax.axis_index("s") * NC + lax.axis_index("c")
    pltpu.sync_copy(perm_hbm, perm_v)
    pltpu.sync_copy(signs_hbm, signs_v)
    base = wid * U_PER_W
    ncb = B // CB
    cols = [jax.lax.iota(jnp.int32, L) + L * j for j in range(CB // L)]

    def hbm_slice(ref, u):
        t = u // ncb
        c = (u % ncb) * CB
        return ref.at[t, :, pl.ds(c, CB)]

    for b in range(NBUF):
        pltpu.async_copy(hbm_slice(x_hbm, base + b), in_bufs[b], in_sems[b])

    def compute(b):
        def row_body(d, _):
            dsplat = jnp.full((L,), d, jnp.int32)
            rsplat = plsc.load_gather(perm_v, [dsplat])
            ssplat = plsc.load_gather(signs_v, [dsplat])
            for j in range(CB // L):
                v = plsc.load_gather(in_bufs[b], [rsplat, cols[j]])
                out_bufs[b][d, pl.ds(L * j, L)] = v * ssplat
            return 0

        plsc.parallel_loop(0, D, 1, unroll=8, carry=jnp.int32(0))(row_body)

    def unit_pair(i2, _):
        for b in range(NBUF):
            u = base + i2 * NBUF + b
            pltpu.make_async_copy(
                hbm_slice(x_hbm, u), in_bufs[b], in_sems[b]).wait()
            @pl.when(i2 > 0)
            def _():
                pltpu.make_async_copy(
                    out_bufs[b], hbm_slice(out_hbm, u - NBUF),
                    out_sems[b]).wait()
            compute(b)
            pltpu.async_copy(out_bufs[b], hbm_slice(out_hbm, u), out_sems[b])
            @pl.when(i2 * NBUF + b + NBUF < U_PER_W)
            def _():
                pltpu.async_copy(
                    hbm_slice(x_hbm, u + NBUF), in_bufs[b], in_sems[b])
        return 0

    lax.fori_loop(0, U_PER_W // NBUF, unit_pair, 0)
    for b in range(NBUF):
        pltpu.make_async_copy(
            out_bufs[b], hbm_slice(out_hbm, base + U_PER_W - NBUF + b),
            out_sems[b]).wait()


@jax.jit
def kernel(x, perm, signs):
    mesh = plsc.VectorSubcoreMesh(
        core_axis_name="c", subcore_axis_name="s", num_cores=NC, num_subcores=NS
    )
    run = pl.kernel(
        _sc_body,
        out_type=jax.ShapeDtypeStruct((T, D, B), jnp.float32),
        mesh=mesh,
        scratch_types=[
            pltpu.VMEM((D,), jnp.int32),
            pltpu.VMEM((D,), jnp.float32),
            pltpu.VMEM((D, CB), jnp.float32),
            pltpu.VMEM((D, CB), jnp.float32),
            pltpu.VMEM((D, CB), jnp.float32),
            pltpu.VMEM((D, CB), jnp.float32),
            pltpu.SemaphoreType.DMA,
            pltpu.SemaphoreType.DMA,
            pltpu.SemaphoreType.DMA,
            pltpu.SemaphoreType.DMA,
        ],
        compiler_params=pltpu.CompilerParams(
            needs_layout_passes=False,
            use_tc_tiling_on_sc=True,
        ),
    )
    yt = run(jnp.transpose(x, (1, 2, 0)), perm, signs)
    return jnp.transpose(yt, (2, 0, 1))


# speedup vs baseline: 10.0820x; 1.1760x over previous
import jax
import jax.numpy as jnp
from jax import lax
from jax.experimental import pallas as pl
from jax.experimental.pallas import tpu as pltpu, tpu_sc as plsc

B = 4096
T = 200
D = 64
NC = 2
NS = 16
NW = NC * NS

CB = 256
NUNITS = T * (B // CB)
U_PER_W = NUNITS // NW
NBUF = 2
L = 16


def _sc_body(x_hbm, perm_hbm, signs_hbm, out_hbm,
             perm_v, signs_v, in_v0, in_v1, out_v0, out_v1,
             in_sem0, in_sem1, out_sem0, out_sem1):
    in_bufs = [in_v0, in_v1]
    out_bufs = [out_v0, out_v1]
    in_sems = [in_sem0, in_sem1]
    out_sems = [out_sem0, out_sem1]
    wid = lax.axis_index("s") * NC + lax.axis_index("c")
    pltpu.sync_copy(perm_hbm, perm_v)
    pltpu.sync_copy(signs_hbm, signs_v)
    base = wid * U_PER_W
    ncb = B // CB
    cols = [jax.lax.iota(jnp.int32, L) + L * j for j in range(CB // L)]

    def hbm_slice(ref, u):
        t = u // ncb
        c = (u % ncb) * CB
        return ref.at[t, :, pl.ds(c, CB)]

    for b in range(NBUF):
        pltpu.async_copy(hbm_slice(x_hbm, base + b), in_bufs[b], in_sems[b])

    def compute(b):
        def row_body(d, _):
            dsplat = jnp.full((L,), d, jnp.int32)
            rsplat = plsc.load_gather(perm_v, [dsplat])
            ssplat = plsc.load_gather(signs_v, [dsplat])
            for j in range(CB // L):
                v = plsc.load_gather(in_bufs[b], [rsplat, cols[j]])
                out_bufs[b][d, pl.ds(L * j, L)] = v * ssplat
            return 0

        plsc.parallel_loop(0, D, 1, unroll=8, carry=jnp.int32(0))(row_body)

    def unit_pair(i2, _):
        for b in range(NBUF):
            u = base + i2 * NBUF + b
            pltpu.make_async_copy(
                hbm_slice(x_hbm, u), in_bufs[b], in_sems[b]).wait()
            @pl.when(i2 > 0)
            def _():
                pltpu.make_async_copy(
                    out_bufs[b], hbm_slice(out_hbm, u - NBUF),
                    out_sems[b]).wait()
            compute(b)
            pltpu.async_copy(out_bufs[b], hbm_slice(out_hbm, u), out_sems[b])
            @pl.when(i2 * NBUF + b + NBUF < U_PER_W)
            def _():
                pltpu.async_copy(
                    hbm_slice(x_hbm, u + NBUF), in_bufs[b], in_sems[b])
        return 0

    lax.fori_loop(0, U_PER_W // NBUF, unit_pair, 0)
    for b in range(NBUF):
        pltpu.make_async_copy(
            out_bufs[b], hbm_slice(out_hbm, base + U_PER_W - NBUF + b),
            out_sems[b]).wait()


@jax.jit
def kernel(x, perm, signs):
    mesh = plsc.VectorSubcoreMesh(
        core_axis_name="c", subcore_axis_name="s", num_cores=NC, num_subcores=NS
    )
    run = pl.kernel(
        _sc_body,
        out_type=jax.ShapeDtypeStruct((T, D, B), jnp.float32),
        mesh=mesh,
        scratch_types=[
            pltpu.VMEM((D,), jnp.int32),
            pltpu.VMEM((D,), jnp.float32),
            pltpu.VMEM((D, CB), jnp.float32),
            pltpu.VMEM((D, CB), jnp.float32),
            pltpu.VMEM((D, CB), jnp.float32),
            pltpu.VMEM((D, CB), jnp.float32),
            pltpu.SemaphoreType.DMA,
            pltpu.SemaphoreType.DMA,
            pltpu.SemaphoreType.DMA,
            pltpu.SemaphoreType.DMA,
        ],
        compiler_params=pltpu.CompilerParams(
            needs_layout_passes=False,
            use_tc_tiling_on_sc=True,
        ),
    )
    yt = run(jnp.transpose(x, (1, 2, 0)), perm, signs)
    return jnp.transpose(yt, (2, 0, 1))


# speedup vs baseline: 10.1380x; 1.0056x over previous
import jax
import jax.numpy as jnp
from jax import lax
from jax.experimental import pallas as pl
from jax.experimental.pallas import tpu as pltpu, tpu_sc as plsc

B = 4096
T = 200
D = 64
NC = 2
NS = 16
NW = NC * NS

CB = 128
NUNITS = T * (B // CB)
U_PER_W = NUNITS // NW
NBUF = 4
L = 16


def _sc_body(x_hbm, perm_hbm, signs_hbm, out_hbm,
             perm_v, signs_v,
             in_v0, in_v1, in_v2, in_v3, out_v0, out_v1, out_v2, out_v3,
             in_sem0, in_sem1, in_sem2, in_sem3,
             out_sem0, out_sem1, out_sem2, out_sem3):
    in_bufs = [in_v0, in_v1, in_v2, in_v3]
    out_bufs = [out_v0, out_v1, out_v2, out_v3]
    in_sems = [in_sem0, in_sem1, in_sem2, in_sem3]
    out_sems = [out_sem0, out_sem1, out_sem2, out_sem3]
    wid = lax.axis_index("s") * NC + lax.axis_index("c")
    pltpu.sync_copy(perm_hbm, perm_v)
    pltpu.sync_copy(signs_hbm, signs_v)
    base = wid * U_PER_W
    ncb = B // CB
    cols = [jax.lax.iota(jnp.int32, L) + L * j for j in range(CB // L)]

    def hbm_slice(ref, u):
        t = u // ncb
        c = (u % ncb) * CB
        return ref.at[t, :, pl.ds(c, CB)]

    for b in range(NBUF):
        pltpu.async_copy(hbm_slice(x_hbm, base + b), in_bufs[b], in_sems[b])

    def compute(b):
        def row_body(d, _):
            dsplat = jnp.full((L,), d, jnp.int32)
            rsplat = plsc.load_gather(perm_v, [dsplat])
            ssplat = plsc.load_gather(signs_v, [dsplat])
            for j in range(CB // L):
                v = plsc.load_gather(in_bufs[b], [rsplat, cols[j]])
                out_bufs[b][d, pl.ds(L * j, L)] = v * ssplat
            return 0

        plsc.parallel_loop(0, D, 1, unroll=8, carry=jnp.int32(0))(row_body)

    def unit_pair(i2, _):
        for b in range(NBUF):
            u = base + i2 * NBUF + b
            pltpu.make_async_copy(
                hbm_slice(x_hbm, u), in_bufs[b], in_sems[b]).wait()
            @pl.when(i2 > 0)
            def _():
                pltpu.make_async_copy(
                    out_bufs[b], hbm_slice(out_hbm, u - NBUF),
                    out_sems[b]).wait()
            compute(b)
            pltpu.async_copy(out_bufs[b], hbm_slice(out_hbm, u), out_sems[b])
            @pl.when(i2 * NBUF + b + NBUF < U_PER_W)
            def _():
                pltpu.async_copy(
                    hbm_slice(x_hbm, u + NBUF), in_bufs[b], in_sems[b])
        return 0

    lax.fori_loop(0, U_PER_W // NBUF, unit_pair, 0)
    for b in range(NBUF):
        pltpu.make_async_copy(
            out_bufs[b], hbm_slice(out_hbm, base + U_PER_W - NBUF + b),
            out_sems[b]).wait()


@jax.jit
def kernel(x, perm, signs):
    mesh = plsc.VectorSubcoreMesh(
        core_axis_name="c", subcore_axis_name="s", num_cores=NC, num_subcores=NS
    )
    run = pl.kernel(
        _sc_body,
        out_type=jax.ShapeDtypeStruct((T, D, B), jnp.float32),
        mesh=mesh,
        scratch_types=[
            pltpu.VMEM((D,), jnp.int32),
            pltpu.VMEM((D,), jnp.float32),
            pltpu.VMEM((D, CB), jnp.float32),
            pltpu.VMEM((D, CB), jnp.float32),
            pltpu.VMEM((D, CB), jnp.float32),
            pltpu.VMEM((D, CB), jnp.float32),
            pltpu.VMEM((D, CB), jnp.float32),
            pltpu.VMEM((D, CB), jnp.float32),
            pltpu.VMEM((D, CB), jnp.float32),
            pltpu.VMEM((D, CB), jnp.float32),
            pltpu.SemaphoreType.DMA,
            pltpu.SemaphoreType.DMA,
            pltpu.SemaphoreType.DMA,
            pltpu.SemaphoreType.DMA,
            pltpu.SemaphoreType.DMA,
            pltpu.SemaphoreType.DMA,
            pltpu.SemaphoreType.DMA,
            pltpu.SemaphoreType.DMA,
        ],
        compiler_params=pltpu.CompilerParams(
            needs_layout_passes=False,
            use_tc_tiling_on_sc=True,
        ),
    )
    yt = run(jnp.transpose(x, (1, 2, 0)), perm, signs)
    return jnp.transpose(yt, (2, 0, 1))
